# baseline (speedup 1.0000x reference)
import functools

import jax
import jax.numpy as jnp
from jax import lax
from jax.experimental import pallas as pl
from jax.experimental.pallas import tpu as pltpu
from jax.experimental.pallas import tpu_sc as plsc

BATCH = 1024
SEQ = 200
EMBED = 128
HALF = SEQ // 2
NC, NS, LANES = 2, 16, 16
NW = NC * NS
ROWS_PER_W = BATCH // NW
VREGS_PER_ROW = EMBED // LANES
NBUF = 4


def _body(idx_hbm, table_hbm, pos_hbm, out_hbm,
          pos_v, idx_v, rows_v,
          gsem0, gsem1, gsem2, gsem3, wsem0, wsem1, wsem2, wsem3):
    gsem = (gsem0, gsem1, gsem2, gsem3)
    wsem = (wsem0, wsem1, wsem2, wsem3)
    wid = lax.axis_index("s") * NC + lax.axis_index("c")
    base = wid * ROWS_PER_W

    pltpu.sync_copy(pos_hbm, pos_v)

    def load_idx(b):
        pltpu.sync_copy(idx_hbm.at[base + b], idx_v.at[b % NBUF])

    def start_gather(b):
        buf = b % NBUF
        return [
            pltpu.async_copy(table_hbm.at[idx_v.at[b % NBUF, h]],
                             rows_v.at[buf, pl.ds(h * HALF, HALF)],
                             gsem[buf], add=True)
            for h in range(2)
        ]

    def prefill(b):
        buf = b % NBUF

        @pl.loop(0, SEQ)
        def _(i):
            for j in range(VREGS_PER_ROW):
                sl = pl.ds(j * LANES, LANES)
                rows_v[buf, i, sl] = pos_v[i, sl]

    load_idx(0)
    load_idx(1)
    prefill(0)
    prefill(1)
    pending_g = {0: start_gather(0), 1: start_gather(1)}
    pending_w = {}
    for b in range(ROWS_PER_W):
        buf = b % NBUF
        if b + 2 < ROWS_PER_W:
            if b >= 2:
                pending_w.pop(b - 2).wait()
            load_idx(b + 2)
            prefill(b + 2)
            pending_g[b + 2] = start_gather(b + 2)
        for d in pending_g.pop(b):
            d.wait()
        pending_w[b] = pltpu.async_copy(rows_v.at[buf], out_hbm.at[base + b],
                                        wsem[buf])
    for b in sorted(pending_w):
        pending_w.pop(b).wait()


@jax.jit
def _embed(idx, token_table, position_table):
    mesh = plsc.VectorSubcoreMesh(core_axis_name="c", subcore_axis_name="s",
                                  num_cores=NC, num_subcores=NS)
    run = pl.kernel(
        _body,
        out_type=jax.ShapeDtypeStruct((BATCH, SEQ, EMBED), jnp.float32),
        mesh=mesh,
        scratch_types=[
            pltpu.VMEM((SEQ, EMBED), jnp.float32),
            pltpu.VMEM((NBUF, 2, HALF), jnp.int32),
            pltpu.VMEM((NBUF, SEQ, EMBED), jnp.float32),
            pltpu.SemaphoreType.DMA,
            pltpu.SemaphoreType.DMA,
            pltpu.SemaphoreType.DMA,
            pltpu.SemaphoreType.DMA,
            pltpu.SemaphoreType.DMA,
            pltpu.SemaphoreType.DMA,
            pltpu.SemaphoreType.DMA,
            pltpu.SemaphoreType.DMA,
        ],
    )
    return run(idx, token_table, position_table)


def kernel(inputs, token_table, position_table):
    idx = inputs.astype(jnp.int32).reshape(BATCH, 2, HALF)
    return _embed(idx, token_table, position_table)
